# Initial kernel scaffold; baseline (speedup 1.0000x reference)
import jax
import jax.numpy as jnp
from jax.experimental import pallas as pl


def kernel(x, edge_index, W1, b1, W2, b2, W3, b3):
    raise NotImplementedError("write your pallas kernel here")


# speedup vs baseline: 2.6295x; 2.6295x over previous
import functools

import jax
import jax.numpy as jnp
from jax import lax
from jax.experimental import pallas as pl
from jax.experimental.pallas import tpu as pltpu
from jax.experimental.pallas import tpu_sc as plsc

NN = 10000
DD = 128
CC = 64
NP = 10240
NC = 2
NS = 16
NW = NC * NS
LL = 16
QR = NP // 4
RPT = QR // NS
EP = 323584
EPW = EP // NW
EPT = EP // NS
CH = 128

_mesh = plsc.VectorSubcoreMesh(core_axis_name="c", subcore_axis_name="s")


@functools.partial(
    pl.kernel,
    out_type=jax.ShapeDtypeStruct((NW, NP), jnp.float32),
    mesh=_mesh,
    scratch_types=[
        pltpu.VMEM((EPW,), jnp.int32),
        pltpu.VMEM((NP,), jnp.float32),
    ],
    compiler_params=pltpu.CompilerParams(needs_layout_passes=False),
)
def _deg_kernel(dst_hbm, out_hbm, idxb, hist):
    c = lax.axis_index("c")
    s = lax.axis_index("s")
    w = s * NC + c

    def initz(j, _):
        hist[pl.ds(j * LL, LL)] = jnp.full((LL,), 0.0, jnp.float32)
        return _

    lax.fori_loop(0, NP // LL, initz, None)
    pltpu.sync_copy(dst_hbm.at[pl.ds(w * EPW, EPW)], idxb)
    ones = jnp.full((LL,), 1.0, jnp.float32)

    def addv(j, _):
        iv = idxb[pl.ds(j * LL, LL)]
        plsc.addupdate_scatter(hist, [iv], ones)
        return _

    lax.fori_loop(0, EPW // LL, addv, None)
    pltpu.sync_copy(hist, out_hbm.at[w])


@functools.partial(
    pl.kernel,
    out_type=jax.ShapeDtypeStruct((NP, DD), jnp.float32),
    mesh=_mesh,
    scratch_types=[
        pltpu.VMEM((CH,), jnp.int32),
        pltpu.VMEM((CH,), jnp.int32),
        pltpu.VMEM((CH,), jnp.int32),
        pltpu.VMEM((CH, DD), jnp.float32),
        pltpu.VMEM((RPT, DD), jnp.float32),
        pltpu.VMEM((RPT, LL), jnp.float32),
        pltpu.VMEM_SHARED((QR + 8, DD), jnp.float32),
        pltpu.SemaphoreType.DMA,
    ],
)
def _pass_kernel(g_hbm, src_hbm, dst_hbm, smat_hbm, out_hbm,
                 srcb, dstb, dlb, rows, obuf, sbuf, acc, sem):
    c = lax.axis_index("c")
    s = lax.axis_index("s")
    ebase = s * EPT
    lbase = s * RPT

    for p in range(2):
        q = 2 * c + p
        qbase = q * QR
        gbase = qbase + s * RPT

        pltpu.sync_copy(g_hbm.at[pl.ds(gbase, RPT)], obuf)
        pltpu.sync_copy(obuf, acc.at[pl.ds(lbase, RPT)])
        plsc.subcore_barrier()

        def chunk(k, _, qbase=qbase):
            off = pl.multiple_of(ebase + k * CH, 8)
            pltpu.sync_copy(src_hbm.at[pl.ds(off, CH)], srcb)
            pltpu.sync_copy(dst_hbm.at[pl.ds(off, CH)], dstb)
            for i in range(CH // LL):
                v = dstb[pl.ds(i * LL, LL)] - qbase
                oob = (v < 0) | (v >= QR)
                dlb[pl.ds(i * LL, LL)] = jnp.where(oob, QR, v)
            pltpu.async_copy(g_hbm.at[srcb], rows, sem).wait()
            pltpu.sync_copy(rows, acc.at[dlb], add=True)
            return _

        lax.fori_loop(0, EPT // CH, chunk, None)
        plsc.subcore_barrier()

        pltpu.sync_copy(smat_hbm.at[pl.ds(gbase, RPT)], sbuf)
        pltpu.sync_copy(acc.at[pl.ds(lbase, RPT)], obuf)

        def scale_row(j, _):
            sv = sbuf[j, :]
            for kk in range(DD // LL):
                obuf[j, pl.ds(kk * LL, LL)] = obuf[j, pl.ds(kk * LL, LL)] * sv
            return _

        lax.fori_loop(0, RPT, scale_row, None)
        pltpu.sync_copy(obuf, out_hbm.at[pl.ds(gbase, RPT)])


_BM = 512


def _tc0_body(x_ref, h_ref, g_ref, dm_ref, d2_ref):
    i = pl.program_id(0)
    deg = jnp.sum(h_ref[...], axis=0)[:, None]
    rid = i * _BM + lax.broadcasted_iota(jnp.int32, (_BM, 1), 0)
    deg = deg + jnp.where(rid < NN, 1.0, 0.0)
    dis = jnp.where(deg > 0, lax.rsqrt(deg), 0.0)
    dm_ref[...] = jnp.broadcast_to(dis, (_BM, LL))
    d2_ref[...] = jnp.broadcast_to(dis * dis, (_BM, LL))
    g_ref[...] = x_ref[...] * dis


def _tc0(xp, hists):
    return pl.pallas_call(
        _tc0_body,
        grid=(NP // _BM,),
        in_specs=[
            pl.BlockSpec((_BM, DD), lambda i: (i, 0)),
            pl.BlockSpec((NW, _BM), lambda i: (0, i)),
        ],
        out_specs=[
            pl.BlockSpec((_BM, DD), lambda i: (i, 0)),
            pl.BlockSpec((_BM, LL), lambda i: (i, 0)),
            pl.BlockSpec((_BM, LL), lambda i: (i, 0)),
        ],
        out_shape=[
            jax.ShapeDtypeStruct((NP, DD), jnp.float32),
            jax.ShapeDtypeStruct((NP, LL), jnp.float32),
            jax.ShapeDtypeStruct((NP, LL), jnp.float32),
        ],
    )(xp, hists)


def _tc1_body(y_ref, w_ref, b_ref, dm_ref, o_ref):
    h = jnp.dot(y_ref[...], w_ref[...], preferred_element_type=jnp.float32)
    h = h + b_ref[...]
    h = jnp.where(h > 0, h, jnp.exp(h) - 1.0)
    o_ref[...] = h * dm_ref[...][:, 0:1]


def _tc1(y, W, b, dm):
    return pl.pallas_call(
        _tc1_body,
        grid=(NP // _BM,),
        in_specs=[
            pl.BlockSpec((_BM, DD), lambda i: (i, 0)),
            pl.BlockSpec((DD, DD), lambda i: (0, 0)),
            pl.BlockSpec((1, DD), lambda i: (0, 0)),
            pl.BlockSpec((_BM, LL), lambda i: (i, 0)),
        ],
        out_specs=pl.BlockSpec((_BM, DD), lambda i: (i, 0)),
        out_shape=jax.ShapeDtypeStruct((NP, DD), jnp.float32),
    )(y, W, b, dm)


def _tc2_body(y_ref, w2_ref, b2_ref, w3_ref, b3_ref, o_ref):
    h = jnp.dot(y_ref[...], w2_ref[...], preferred_element_type=jnp.float32)
    h = h + b2_ref[...]
    h = jnp.where(h > 0, h, jnp.exp(h) - 1.0)
    o = jnp.dot(h, w3_ref[...], preferred_element_type=jnp.float32)
    o = o + b3_ref[...]
    o_ref[...] = jnp.maximum(o, 0.0)


def _tc2(y, W2, b2, W3, b3):
    return pl.pallas_call(
        _tc2_body,
        grid=(NP // _BM,),
        in_specs=[
            pl.BlockSpec((_BM, DD), lambda i: (i, 0)),
            pl.BlockSpec((DD, DD), lambda i: (0, 0)),
            pl.BlockSpec((1, DD), lambda i: (0, 0)),
            pl.BlockSpec((DD, CC), lambda i: (0, 0)),
            pl.BlockSpec((1, CC), lambda i: (0, 0)),
        ],
        out_specs=pl.BlockSpec((_BM, CC), lambda i: (i, 0)),
        out_shape=jax.ShapeDtypeStruct((NP, CC), jnp.float32),
    )(y, W2, b2, W3, b3)


def kernel(x, edge_index, W1, b1, W2, b2, W3, b3):
    E = edge_index.shape[1]
    src = edge_index[0]
    dst = edge_index[1]
    pad = jnp.full((EP - E,), NP - 1, jnp.int32)
    srcp = jnp.concatenate([src, pad])
    dstp = jnp.concatenate([dst, pad])
    xp = jnp.pad(x, ((0, NP - NN), (0, 0)))

    hists = _deg_kernel(dstp)
    g0, dism, dis2m = _tc0(xp, hists)
    q = _pass_kernel(g0, srcp, dstp, dis2m)
    y1 = _pass_kernel(q, srcp, dstp, dism)
    g1 = _tc1(y1, W1, b1.reshape(1, DD), dism)
    q = _pass_kernel(g1, srcp, dstp, dis2m)
    y2 = _pass_kernel(q, srcp, dstp, dism)
    out = _tc2(y2, W2, b2.reshape(1, DD), W3, b3.reshape(1, CC))
    return out[:NN]


# speedup vs baseline: 3.2041x; 1.2185x over previous
import functools

import jax
import jax.numpy as jnp
from jax import lax
from jax.experimental import pallas as pl
from jax.experimental.pallas import tpu as pltpu
from jax.experimental.pallas import tpu_sc as plsc

NN = 10000
DD = 128
CC = 64
NP = 10240
NC = 2
NS = 16
NW = NC * NS
LL = 16
QR = NP // 4
RPT = QR // NS
EP = 323584
EPW = EP // NW
EPT = EP // NS
CH = 128

_mesh = plsc.VectorSubcoreMesh(core_axis_name="c", subcore_axis_name="s")


@functools.partial(
    pl.kernel,
    out_type=jax.ShapeDtypeStruct((NW, NP), jnp.float32),
    mesh=_mesh,
    scratch_types=[
        pltpu.VMEM((EPW,), jnp.int32),
        pltpu.VMEM((NP,), jnp.float32),
    ],
    compiler_params=pltpu.CompilerParams(needs_layout_passes=False),
)
def _deg_kernel(dst_hbm, out_hbm, idxb, hist):
    c = lax.axis_index("c")
    s = lax.axis_index("s")
    w = s * NC + c

    def initz(j, _):
        hist[pl.ds(j * LL, LL)] = jnp.full((LL,), 0.0, jnp.float32)
        return _

    lax.fori_loop(0, NP // LL, initz, None)
    pltpu.sync_copy(dst_hbm.at[pl.ds(w * EPW, EPW)], idxb)
    ones = jnp.full((LL,), 1.0, jnp.float32)

    def addv(j, _):
        iv = idxb[pl.ds(j * LL, LL)]
        plsc.addupdate_scatter(hist, [iv], ones)
        return _

    lax.fori_loop(0, EPW // LL, addv, None)
    pltpu.sync_copy(hist, out_hbm.at[w])


@functools.partial(
    pl.kernel,
    out_type=jax.ShapeDtypeStruct((NP, DD), jnp.float32),
    mesh=_mesh,
    scratch_types=[
        pltpu.VMEM((CH,), jnp.int32),
        pltpu.VMEM((CH,), jnp.int32),
        pltpu.VMEM((CH,), jnp.int32),
        pltpu.VMEM((CH,), jnp.int32),
        pltpu.VMEM((CH,), jnp.int32),
        pltpu.VMEM((CH,), jnp.int32),
        pltpu.VMEM((CH, DD), jnp.float32),
        pltpu.VMEM((CH, DD), jnp.float32),
        pltpu.VMEM((RPT, DD), jnp.float32),
        pltpu.VMEM((RPT, LL), jnp.float32),
        pltpu.VMEM_SHARED((QR + 8, DD), jnp.float32),
        pltpu.SemaphoreType.DMA,
        pltpu.SemaphoreType.DMA,
    ],
)
def _pass_kernel(g_hbm, src_hbm, dst_hbm, smat_hbm, out_hbm,
                 srcb0, dstb0, dlb0, srcb1, dstb1, dlb1,
                 rows0, rows1, obuf, sbuf, acc, sem0, sem1):
    c = lax.axis_index("c")
    s = lax.axis_index("s")
    ebase = s * EPT
    lbase = s * RPT
    npair = EPT // CH // 2

    for p in range(2):
        q = 2 * c + p
        qbase = q * QR
        gbase = qbase + s * RPT

        pltpu.sync_copy(g_hbm.at[pl.ds(gbase, RPT)], obuf)
        pltpu.sync_copy(obuf, acc.at[pl.ds(lbase, RPT)])
        plsc.subcore_barrier()

        def prep(k, srcb, dstb, dlb, qbase=qbase):
            off = pl.multiple_of(ebase + k * CH, 8)
            pltpu.sync_copy(src_hbm.at[pl.ds(off, CH)], srcb)
            pltpu.sync_copy(dst_hbm.at[pl.ds(off, CH)], dstb)
            for i in range(CH // LL):
                v = dstb[pl.ds(i * LL, LL)] - qbase
                oob = (v < 0) | (v >= QR)
                dlb[pl.ds(i * LL, LL)] = jnp.where(oob, QR, v)

        prep(0, srcb0, dstb0, dlb0)
        pltpu.async_copy(g_hbm.at[srcb0], rows0, sem0)

        def pair(kk, _):
            prep(2 * kk + 1, srcb1, dstb1, dlb1)
            pltpu.async_copy(g_hbm.at[srcb1], rows1, sem1)
            pltpu.make_async_copy(g_hbm.at[srcb0], rows0, sem0).wait()
            pltpu.sync_copy(rows0, acc.at[dlb0], add=True)

            @pl.when(kk < npair - 1)
            def _():
                prep(2 * kk + 2, srcb0, dstb0, dlb0)
                pltpu.async_copy(g_hbm.at[srcb0], rows0, sem0)

            pltpu.make_async_copy(g_hbm.at[srcb1], rows1, sem1).wait()
            pltpu.sync_copy(rows1, acc.at[dlb1], add=True)
            return _

        lax.fori_loop(0, npair, pair, None)
        plsc.subcore_barrier()

        pltpu.sync_copy(smat_hbm.at[pl.ds(gbase, RPT)], sbuf)
        pltpu.sync_copy(acc.at[pl.ds(lbase, RPT)], obuf)

        def scale_row(j, _):
            sv = sbuf[j, :]
            for kk in range(DD // LL):
                obuf[j, pl.ds(kk * LL, LL)] = obuf[j, pl.ds(kk * LL, LL)] * sv
            return _

        lax.fori_loop(0, RPT, scale_row, None)
        pltpu.sync_copy(obuf, out_hbm.at[pl.ds(gbase, RPT)])


_BM = 512


def _tc0_body(x_ref, h_ref, g_ref, dm_ref, d2_ref):
    i = pl.program_id(0)
    deg = jnp.sum(h_ref[...], axis=0)[:, None]
    rid = i * _BM + lax.broadcasted_iota(jnp.int32, (_BM, 1), 0)
    deg = deg + jnp.where(rid < NN, 1.0, 0.0)
    dis = jnp.where(deg > 0, lax.rsqrt(deg), 0.0)
    dm_ref[...] = jnp.broadcast_to(dis, (_BM, LL))
    d2_ref[...] = jnp.broadcast_to(dis * dis, (_BM, LL))
    g_ref[...] = x_ref[...] * dis


def _tc0(xp, hists):
    return pl.pallas_call(
        _tc0_body,
        grid=(NP // _BM,),
        in_specs=[
            pl.BlockSpec((_BM, DD), lambda i: (i, 0)),
            pl.BlockSpec((NW, _BM), lambda i: (0, i)),
        ],
        out_specs=[
            pl.BlockSpec((_BM, DD), lambda i: (i, 0)),
            pl.BlockSpec((_BM, LL), lambda i: (i, 0)),
            pl.BlockSpec((_BM, LL), lambda i: (i, 0)),
        ],
        out_shape=[
            jax.ShapeDtypeStruct((NP, DD), jnp.float32),
            jax.ShapeDtypeStruct((NP, LL), jnp.float32),
            jax.ShapeDtypeStruct((NP, LL), jnp.float32),
        ],
    )(xp, hists)


def _tc1_body(y_ref, w_ref, b_ref, dm_ref, o_ref):
    h = jnp.dot(y_ref[...], w_ref[...], preferred_element_type=jnp.float32)
    h = h + b_ref[...]
    h = jnp.where(h > 0, h, jnp.exp(h) - 1.0)
    o_ref[...] = h * dm_ref[...][:, 0:1]


def _tc1(y, W, b, dm):
    return pl.pallas_call(
        _tc1_body,
        grid=(NP // _BM,),
        in_specs=[
            pl.BlockSpec((_BM, DD), lambda i: (i, 0)),
            pl.BlockSpec((DD, DD), lambda i: (0, 0)),
            pl.BlockSpec((1, DD), lambda i: (0, 0)),
            pl.BlockSpec((_BM, LL), lambda i: (i, 0)),
        ],
        out_specs=pl.BlockSpec((_BM, DD), lambda i: (i, 0)),
        out_shape=jax.ShapeDtypeStruct((NP, DD), jnp.float32),
    )(y, W, b, dm)


def _tc2_body(y_ref, w2_ref, b2_ref, w3_ref, b3_ref, o_ref):
    h = jnp.dot(y_ref[...], w2_ref[...], preferred_element_type=jnp.float32)
    h = h + b2_ref[...]
    h = jnp.where(h > 0, h, jnp.exp(h) - 1.0)
    o = jnp.dot(h, w3_ref[...], preferred_element_type=jnp.float32)
    o = o + b3_ref[...]
    o_ref[...] = jnp.maximum(o, 0.0)


def _tc2(y, W2, b2, W3, b3):
    return pl.pallas_call(
        _tc2_body,
        grid=(NP // _BM,),
        in_specs=[
            pl.BlockSpec((_BM, DD), lambda i: (i, 0)),
            pl.BlockSpec((DD, DD), lambda i: (0, 0)),
            pl.BlockSpec((1, DD), lambda i: (0, 0)),
            pl.BlockSpec((DD, CC), lambda i: (0, 0)),
            pl.BlockSpec((1, CC), lambda i: (0, 0)),
        ],
        out_specs=pl.BlockSpec((_BM, CC), lambda i: (i, 0)),
        out_shape=jax.ShapeDtypeStruct((NP, CC), jnp.float32),
    )(y, W2, b2, W3, b3)


def kernel(x, edge_index, W1, b1, W2, b2, W3, b3):
    E = edge_index.shape[1]
    src = edge_index[0]
    dst = edge_index[1]
    pad = jnp.full((EP - E,), NP - 1, jnp.int32)
    srcp = jnp.concatenate([src, pad])
    dstp = jnp.concatenate([dst, pad])
    xp = jnp.pad(x, ((0, NP - NN), (0, 0)))

    hists = _deg_kernel(dstp)
    g0, dism, dis2m = _tc0(xp, hists)
    q = _pass_kernel(g0, srcp, dstp, dis2m)
    y1 = _pass_kernel(q, srcp, dstp, dism)
    g1 = _tc1(y1, W1, b1.reshape(1, DD), dism)
    q = _pass_kernel(g1, srcp, dstp, dis2m)
    y2 = _pass_kernel(q, srcp, dstp, dism)
    out = _tc2(y2, W2, b2.reshape(1, DD), W3, b3.reshape(1, CC))
    return out[:NN]
